# Initial kernel scaffold; baseline (speedup 1.0000x reference)
import functools

import jax
import jax.numpy as jnp
from jax import lax
from jax.experimental import pallas as pl
from jax.experimental.pallas import tpu as pltpu
from jax.experimental.pallas import tpu_sc as plsc

VOCAB = 100000
EMBED_DIM = 128
BATCH = 16384
SEQ = 200

SEQP = 208
VB = 512
VP = ((VOCAB + 8 + VB - 1) // VB) * VB
BIAS_IDX = VOCAB
ZERO_IDX = VOCAB + 1

NC = 2
NS = 16
NW = NC * NS
RPP = BATCH // NS
CH = 64
NCHUNK = RPP // CH


def _proj_body(w_ref, b_ref, emb_ref, o_ref):
    i = pl.program_id(0)
    acc = lax.dot_general(
        w_ref[...], emb_ref[...],
        (((1,), (1,)), ((), ())),
        preferred_element_type=jnp.float32,
    )
    v = i * VB + lax.broadcasted_iota(jnp.int32, (8, VB), 1)
    o_ref[...] = jnp.where(v < VOCAB, acc,
                           jnp.where(v == BIAS_IDX, b_ref[...], 0.0))


def _project_table(emb_table, fc_w, fc_b):
    w8 = jnp.zeros((8, EMBED_DIM), jnp.float32).at[:2].set(fc_w * (1.0 / SEQ))
    b8 = jnp.zeros((8,), jnp.float32).at[:2].set(fc_b)
    b_full = jnp.broadcast_to(b8[:, None], (8, VB))
    return pl.pallas_call(
        _proj_body,
        grid=(VP // VB,),
        in_specs=[
            pl.BlockSpec((8, EMBED_DIM), lambda i: (0, 0)),
            pl.BlockSpec((8, VB), lambda i: (0, 0)),
            pl.BlockSpec((VB, EMBED_DIM), lambda i: (i, 0)),
        ],
        out_specs=pl.BlockSpec((8, VB), lambda i: (0, i)),
        out_shape=jax.ShapeDtypeStruct((8, VP), jnp.float32),
    )(w8, b_full, emb_table)


def _sc_pool_body(p8_hbm, x_hbm, out_hbm, p_v, xi_v, o_v):
    wid = lax.axis_index("s") * NC + lax.axis_index("c")
    cls = wid // NS
    part = wid % NS
    pltpu.sync_copy(p8_hbm.at[cls], p_v)
    row0 = part * RPP

    def chunk_body(ch, carry):
        base = (row0 + ch * CH) * SEQP
        pltpu.sync_copy(x_hbm.at[pl.ds(base, CH * SEQP)], xi_v)

        def row_body(r, carry2):
            off = r * SEQP
            acc = jnp.zeros((16,), jnp.float32)
            for j in range(SEQP // 16):
                idx = xi_v[pl.ds(off + j * 16, 16)]
                acc = acc + plsc.load_gather(p_v, [idx])
            o_v[ch * CH + r] = jnp.sum(acc)
            return carry2

        lax.fori_loop(0, CH, row_body, 0)
        return carry

    lax.fori_loop(0, NCHUNK, chunk_body, 0)
    pltpu.sync_copy(o_v, out_hbm.at[cls, pl.ds(row0, RPP)])


_sc_pool = functools.partial(
    pl.kernel,
    out_type=jax.ShapeDtypeStruct((2, BATCH), jnp.float32),
    mesh=plsc.VectorSubcoreMesh(core_axis_name="c", subcore_axis_name="s"),
    scratch_types=[
        pltpu.VMEM((VP,), jnp.float32),
        pltpu.VMEM((CH * SEQP,), jnp.int32),
        pltpu.VMEM((RPP,), jnp.float32),
    ],
)(_sc_pool_body)


@jax.jit
def kernel(x, emb_table, fc_w, fc_b):
    p8 = _project_table(emb_table, fc_w, fc_b)
    pad = jnp.full((BATCH, SEQP - SEQ), ZERO_IDX, jnp.int32).at[:, 0].set(BIAS_IDX)
    x_flat = jnp.concatenate([x.astype(jnp.int32), pad], axis=1).reshape(-1)
    out2 = _sc_pool(p8, x_flat)
    return out2.T


# speedup vs baseline: 64.9698x; 64.9698x over previous
import functools

import jax
import jax.numpy as jnp
from jax import lax
from jax.experimental import pallas as pl
from jax.experimental.pallas import tpu as pltpu
from jax.experimental.pallas import tpu_sc as plsc

VOCAB = 100000
EMBED_DIM = 128
BATCH = 16384
SEQ = 200

SEQP = 208
VB = 512
VP = ((VOCAB + 8 + VB - 1) // VB) * VB
BIAS_IDX = VOCAB
ZERO_IDX = VOCAB + 1

NC = 2
NS = 16
NW = NC * NS
RPP = BATCH // NS
CH = 64
NCHUNK = RPP // CH


def _proj_body(w_ref, b_ref, emb_ref, o_ref):
    i = pl.program_id(0)
    acc = lax.dot_general(
        w_ref[...], emb_ref[...],
        (((1,), (1,)), ((), ())),
        preferred_element_type=jnp.float32,
    )
    v = i * VB + lax.broadcasted_iota(jnp.int32, (8, VB), 1)
    o_ref[...] = jnp.where(v < VOCAB, acc,
                           jnp.where(v == BIAS_IDX, b_ref[...], 0.0))


def _project_table(emb_table, fc_w, fc_b):
    w8 = jnp.zeros((8, EMBED_DIM), jnp.float32).at[:2].set(fc_w * (1.0 / SEQ))
    b8 = jnp.zeros((8,), jnp.float32).at[:2].set(fc_b)
    b_full = jnp.broadcast_to(b8[:, None], (8, VB))
    return pl.pallas_call(
        _proj_body,
        grid=(VP // VB,),
        in_specs=[
            pl.BlockSpec((8, EMBED_DIM), lambda i: (0, 0)),
            pl.BlockSpec((8, VB), lambda i: (0, 0)),
            pl.BlockSpec((VB, EMBED_DIM), lambda i: (i, 0)),
        ],
        out_specs=pl.BlockSpec((8, VB), lambda i: (0, i)),
        out_shape=jax.ShapeDtypeStruct((8, VP), jnp.float32),
    )(w8, b_full, emb_table)


def _sc_pool_body(p8_hbm, x_hbm, out_hbm, p_v, xi_v, o_v):
    wid = lax.axis_index("s") * NC + lax.axis_index("c")
    cls = wid // NS
    part = wid % NS
    pltpu.sync_copy(p8_hbm.at[cls], p_v)
    row0 = part * RPP

    lane = lax.broadcasted_iota(jnp.int32, (16,), 0)

    def chunk_body(ch, carry):
        base = (row0 + ch * CH) * SEQP
        pltpu.sync_copy(x_hbm.at[pl.ds(base, CH * SEQP)], xi_v)

        def group_body(g, carry2):
            def row_body(rr, vec):
                off = (g * 16 + rr) * SEQP
                acc = jnp.zeros((16,), jnp.float32)
                for j in range(SEQP // 16):
                    idx = xi_v[pl.ds(off + j * 16, 16)]
                    acc = acc + plsc.load_gather(p_v, [idx])
                return jnp.where(lane == rr, jnp.sum(acc), vec)

            vec = lax.fori_loop(0, 16, row_body, jnp.zeros((16,), jnp.float32))
            o_v[pl.ds(ch * CH + g * 16, 16)] = vec
            return carry2

        lax.fori_loop(0, CH // 16, group_body, 0)
        return carry

    lax.fori_loop(0, NCHUNK, chunk_body, 0)
    pltpu.sync_copy(o_v, out_hbm.at[cls, pl.ds(row0, RPP)])


_sc_pool = functools.partial(
    pl.kernel,
    out_type=jax.ShapeDtypeStruct((2, BATCH), jnp.float32),
    mesh=plsc.VectorSubcoreMesh(
        core_axis_name="c", subcore_axis_name="s",
        num_cores=NC, num_subcores=NS,
    ),
    scratch_types=[
        pltpu.VMEM((VP,), jnp.float32),
        pltpu.VMEM((CH * SEQP,), jnp.int32),
        pltpu.VMEM((RPP,), jnp.float32),
    ],
    compiler_params=pltpu.CompilerParams(needs_layout_passes=False),
)(_sc_pool_body)


@jax.jit
def kernel(x, emb_table, fc_w, fc_b):
    p8 = _project_table(emb_table, fc_w, fc_b)
    pad = jnp.full((BATCH, SEQP - SEQ), ZERO_IDX, jnp.int32).at[:, 0].set(BIAS_IDX)
    x_flat = jnp.concatenate([x.astype(jnp.int32), pad], axis=1).reshape(-1)
    out2 = _sc_pool(p8, x_flat)
    return out2.T


# speedup vs baseline: 76.5799x; 1.1787x over previous
import functools

import jax
import jax.numpy as jnp
from jax import lax
from jax.experimental import pallas as pl
from jax.experimental.pallas import tpu as pltpu
from jax.experimental.pallas import tpu_sc as plsc

VOCAB = 100000
EMBED_DIM = 128
BATCH = 16384
SEQ = 200

VB = 512
VP = ((VOCAB + 8 + VB - 1) // VB) * VB
BIAS_IDX = VOCAB
VR = VOCAB + 8

NC = 2
NS = 16
NW = NC * NS
RPT = BATCH // NW
CH = 64
NCH = RPT // CH
CHW = CH * SEQ
NFULL = SEQ // 16
TAIL = SEQ - NFULL * 16


def _proj_body(w_ref, pb_ref, emb_ref, o_ref):
    i = pl.program_id(0)
    acc = lax.dot_general(
        w_ref[...], emb_ref[...],
        (((1,), (1,)), ((), ())),
        preferred_element_type=jnp.float32,
    )
    u0 = lax.bitcast_convert_type(
        acc[0:1, :].astype(jnp.bfloat16), jnp.uint16).astype(jnp.int32)
    u1 = lax.bitcast_convert_type(
        acc[1:2, :].astype(jnp.bfloat16), jnp.uint16).astype(jnp.int32)
    packed = u0 | (u1 << 16)
    v = i * VB + lax.broadcasted_iota(jnp.int32, (1, VB), 1)
    o_ref[...] = jnp.where(v < VOCAB, packed,
                           jnp.where(v == BIAS_IDX, pb_ref[...], 0))


def _project_table(emb_table, fc_w, fc_b):
    w8 = jnp.zeros((8, EMBED_DIM), jnp.float32).at[:2].set(fc_w * (1.0 / SEQ))
    bu = lax.bitcast_convert_type(
        fc_b.astype(jnp.bfloat16), jnp.uint16).astype(jnp.int32)
    pbias = bu[0] | (bu[1] << 16)
    pb_full = jnp.broadcast_to(pbias, (1, VB))
    return pl.pallas_call(
        _proj_body,
        grid=(VP // VB,),
        in_specs=[
            pl.BlockSpec((8, EMBED_DIM), lambda i: (0, 0)),
            pl.BlockSpec((1, VB), lambda i: (0, 0)),
            pl.BlockSpec((VB, EMBED_DIM), lambda i: (i, 0)),
        ],
        out_specs=pl.BlockSpec((1, VB), lambda i: (0, i)),
        out_shape=jax.ShapeDtypeStruct((1, VP), jnp.int32),
    )(w8, pb_full, emb_table).reshape(VP)


def _sc_pool_body(p_hbm, x_hbm, out_hbm,
                  p_v, xa_v, xb_v, o_v, sem_p, sem_a, sem_b):
    wid = lax.axis_index("s") * NC + lax.axis_index("c")
    row0 = wid * RPT

    cp_p = pltpu.async_copy(p_hbm.at[pl.ds(0, VR)], p_v, sem_p)
    xa_v[pl.ds(CHW, 16)] = jnp.zeros((16,), jnp.int32)
    xb_v[pl.ds(CHW, 16)] = jnp.zeros((16,), jnp.int32)

    bufs = (xa_v, xb_v)
    sems = (sem_a, sem_b)
    handles = [None] * NCH
    handles[0] = pltpu.async_copy(
        x_hbm.at[pl.ds(row0 * SEQ, CHW)], xa_v.at[pl.ds(0, CHW)], sem_a)

    cp_p.wait()
    bvec = plsc.load_gather(p_v, [jnp.full((16,), BIAS_IDX, jnp.int32)])
    b0s, b1s = plsc.unpack(plsc.bitcast(bvec, jnp.bfloat16),
                           format=plsc.PackFormat.INTERLEAVED)
    lane = lax.broadcasted_iota(jnp.int32, (16,), 0)
    tail_mask = lane < TAIL
    zf = jnp.zeros((16,), jnp.float32)

    for ch in range(NCH):
        if ch + 1 < NCH:
            handles[ch + 1] = pltpu.async_copy(
                x_hbm.at[pl.ds((row0 + (ch + 1) * CH) * SEQ, CHW)],
                bufs[(ch + 1) % 2].at[pl.ds(0, CHW)], sems[(ch + 1) % 2])
        handles[ch].wait()
        cur = bufs[ch % 2]

        def group_body(g, carry, cur=cur, ch=ch):
            def row_body(rr, vecs):
                vec0, vec1 = vecs
                off = (g * 16 + rr) * SEQ
                acc0 = zf
                acc1 = zf
                for j in range(NFULL):
                    gi = plsc.load_gather(p_v, [cur[pl.ds(off + j * 16, 16)]])
                    a, b = plsc.unpack(plsc.bitcast(gi, jnp.bfloat16),
                                       format=plsc.PackFormat.INTERLEAVED)
                    acc0 = acc0 + a
                    acc1 = acc1 + b
                gi = plsc.load_gather(p_v, [cur[pl.ds(off + NFULL * 16, 16)]])
                a, b = plsc.unpack(plsc.bitcast(gi, jnp.bfloat16),
                                   format=plsc.PackFormat.INTERLEAVED)
                acc0 = acc0 + jnp.where(tail_mask, a, 0.0)
                acc1 = acc1 + jnp.where(tail_mask, b, 0.0)
                return (jnp.where(lane == rr, jnp.sum(acc0), vec0),
                        jnp.where(lane == rr, jnp.sum(acc1), vec1))

            vec0, vec1 = lax.fori_loop(0, 16, row_body, (zf, zf))
            base = ch * CH + g * 16
            o_v[pl.ds(base, 16)] = vec0 + b0s
            o_v[pl.ds(RPT + base, 16)] = vec1 + b1s
            return carry

        lax.fori_loop(0, CH // 16, group_body, 0)

    pltpu.sync_copy(o_v.at[pl.ds(0, RPT)], out_hbm.at[0, pl.ds(row0, RPT)])
    pltpu.sync_copy(o_v.at[pl.ds(RPT, RPT)], out_hbm.at[1, pl.ds(row0, RPT)])


_sc_pool = functools.partial(
    pl.kernel,
    out_type=jax.ShapeDtypeStruct((2, BATCH), jnp.float32),
    mesh=plsc.VectorSubcoreMesh(
        core_axis_name="c", subcore_axis_name="s",
        num_cores=NC, num_subcores=NS,
    ),
    scratch_types=[
        pltpu.VMEM((VR,), jnp.int32),
        pltpu.VMEM((CHW + 16,), jnp.int32),
        pltpu.VMEM((CHW + 16,), jnp.int32),
        pltpu.VMEM((2 * RPT,), jnp.float32),
        pltpu.SemaphoreType.DMA,
        pltpu.SemaphoreType.DMA,
        pltpu.SemaphoreType.DMA,
    ],
    compiler_params=pltpu.CompilerParams(needs_layout_passes=False),
)(_sc_pool_body)


@jax.jit
def kernel(x, emb_table, fc_w, fc_b):
    p_packed = _project_table(emb_table, fc_w, fc_b)
    x_flat = x.astype(jnp.int32).reshape(BATCH * SEQ)
    out2 = _sc_pool(p_packed, x_flat)
    return out2.T


# speedup vs baseline: 141.8641x; 1.8525x over previous
import functools

import jax
import jax.numpy as jnp
from jax import lax
from jax.experimental import pallas as pl
from jax.experimental.pallas import tpu as pltpu
from jax.experimental.pallas import tpu_sc as plsc

VOCAB = 100000
EMBED_DIM = 128
BATCH = 16384
SEQ = 200

VB = 4096
VP = ((VOCAB + 8 + VB - 1) // VB) * VB
BIAS_IDX = VOCAB
VR = VOCAB + 8

NC = 2
NS = 16
NW = NC * NS
RPT = BATCH // NW
CH = 64
NCH = RPT // CH
CHW = CH * SEQ
NFULL = SEQ // 16
TAIL = SEQ - NFULL * 16


def _proj_body(w_ref, pb_ref, emb_ref, o_ref):
    i = pl.program_id(0)
    acc = lax.dot_general(
        w_ref[...], emb_ref[...],
        (((1,), (1,)), ((), ())),
        preferred_element_type=jnp.float32,
    )
    u0 = lax.bitcast_convert_type(
        acc[0:1, :].astype(jnp.bfloat16), jnp.uint16).astype(jnp.int32)
    u1 = lax.bitcast_convert_type(
        acc[1:2, :].astype(jnp.bfloat16), jnp.uint16).astype(jnp.int32)
    packed = u0 | (u1 << 16)
    v = i * VB + lax.broadcasted_iota(jnp.int32, (1, VB), 1)
    o_ref[...] = jnp.where(v < VOCAB, packed,
                           jnp.where(v == BIAS_IDX, pb_ref[...], 0))


def _project_table(emb_table, fc_w, fc_b):
    w8 = jnp.zeros((8, EMBED_DIM), jnp.float32).at[:2].set(fc_w * (1.0 / SEQ))
    bu = lax.bitcast_convert_type(
        fc_b.astype(jnp.bfloat16), jnp.uint16).astype(jnp.int32)
    pbias = bu[0] | (bu[1] << 16)
    pb_full = jnp.broadcast_to(pbias, (1, VB))
    return pl.pallas_call(
        _proj_body,
        grid=(VP // VB,),
        in_specs=[
            pl.BlockSpec((8, EMBED_DIM), lambda i: (0, 0)),
            pl.BlockSpec((1, VB), lambda i: (0, 0)),
            pl.BlockSpec((VB, EMBED_DIM), lambda i: (i, 0)),
        ],
        out_specs=pl.BlockSpec((1, VB), lambda i: (0, i)),
        out_shape=jax.ShapeDtypeStruct((1, VP), jnp.int32),
    )(w8, pb_full, emb_table).reshape(VP)


def _sc_pool_body(p_hbm, x_hbm, out_hbm,
                  p_v, xa_v, xb_v, o_v, sem_p, sem_a, sem_b):
    wid = lax.axis_index("s") * NC + lax.axis_index("c")
    row0 = wid * RPT

    cp_p = pltpu.async_copy(p_hbm.at[pl.ds(0, VR)], p_v, sem_p)
    xa_v[pl.ds(CHW, 16)] = jnp.zeros((16,), jnp.int32)
    xb_v[pl.ds(CHW, 16)] = jnp.zeros((16,), jnp.int32)

    bufs = (xa_v, xb_v)
    sems = (sem_a, sem_b)
    handles = [None] * NCH
    handles[0] = pltpu.async_copy(
        x_hbm.at[pl.ds(row0 * SEQ, CHW)], xa_v.at[pl.ds(0, CHW)], sem_a)

    cp_p.wait()
    bvec = plsc.load_gather(p_v, [jnp.full((16,), BIAS_IDX, jnp.int32)])
    b0s, b1s = plsc.unpack(plsc.bitcast(bvec, jnp.bfloat16),
                           format=plsc.PackFormat.INTERLEAVED)
    lane = lax.broadcasted_iota(jnp.int32, (16,), 0)
    tail_mask = lane < TAIL
    zf = jnp.zeros((16,), jnp.float32)

    for ch in range(NCH):
        if ch + 1 < NCH:
            handles[ch + 1] = pltpu.async_copy(
                x_hbm.at[pl.ds((row0 + (ch + 1) * CH) * SEQ, CHW)],
                bufs[(ch + 1) % 2].at[pl.ds(0, CHW)], sems[(ch + 1) % 2])
        handles[ch].wait()
        cur = bufs[ch % 2]

        def group_body(g, carry, cur=cur, ch=ch):
            def row_body(rr, vecs):
                vec0, vec1 = vecs
                off = (g * 16 + rr) * SEQ
                acc0 = zf
                acc1 = zf
                for j in range(NFULL):
                    gi = plsc.load_gather(p_v, [cur[pl.ds(off + j * 16, 16)]])
                    a, b = plsc.unpack(plsc.bitcast(gi, jnp.bfloat16),
                                       format=plsc.PackFormat.INTERLEAVED)
                    acc0 = acc0 + a
                    acc1 = acc1 + b
                gi = plsc.load_gather(p_v, [cur[pl.ds(off + NFULL * 16, 16)]])
                a, b = plsc.unpack(plsc.bitcast(gi, jnp.bfloat16),
                                   format=plsc.PackFormat.INTERLEAVED)
                acc0 = acc0 + jnp.where(tail_mask, a, 0.0)
                acc1 = acc1 + jnp.where(tail_mask, b, 0.0)
                return (jnp.where(lane == rr, jnp.sum(acc0), vec0),
                        jnp.where(lane == rr, jnp.sum(acc1), vec1))

            vec0, vec1 = lax.fori_loop(0, 16, row_body, (zf, zf))
            base = ch * CH + g * 16
            o_v[pl.ds(base, 16)] = vec0 + b0s
            o_v[pl.ds(RPT + base, 16)] = vec1 + b1s
            return carry

        lax.fori_loop(0, CH // 16, group_body, 0)

    pltpu.sync_copy(o_v.at[pl.ds(0, RPT)], out_hbm.at[0, pl.ds(row0, RPT)])
    pltpu.sync_copy(o_v.at[pl.ds(RPT, RPT)], out_hbm.at[1, pl.ds(row0, RPT)])


_sc_pool = functools.partial(
    pl.kernel,
    out_type=jax.ShapeDtypeStruct((2, BATCH), jnp.float32),
    mesh=plsc.VectorSubcoreMesh(
        core_axis_name="c", subcore_axis_name="s",
        num_cores=NC, num_subcores=NS,
    ),
    scratch_types=[
        pltpu.VMEM((VR,), jnp.int32),
        pltpu.VMEM((CHW + 16,), jnp.int32),
        pltpu.VMEM((CHW + 16,), jnp.int32),
        pltpu.VMEM((2 * RPT,), jnp.float32),
        pltpu.SemaphoreType.DMA,
        pltpu.SemaphoreType.DMA,
        pltpu.SemaphoreType.DMA,
    ],
    compiler_params=pltpu.CompilerParams(needs_layout_passes=False),
)(_sc_pool_body)


@jax.jit
def kernel(x, emb_table, fc_w, fc_b):
    p_packed = _project_table(emb_table, fc_w, fc_b)
    x_flat = x.astype(jnp.int32).reshape(BATCH * SEQ)
    out2 = _sc_pool(p_packed, x_flat)
    return out2.T


# speedup vs baseline: 148.9858x; 1.0502x over previous
import functools

import jax
import jax.numpy as jnp
from jax import lax
from jax.experimental import pallas as pl
from jax.experimental.pallas import tpu as pltpu
from jax.experimental.pallas import tpu_sc as plsc

VOCAB = 100000
EMBED_DIM = 128
BATCH = 16384
SEQ = 200

VB = 8192
VP = ((VOCAB + 8 + VB - 1) // VB) * VB
BIAS_IDX = VOCAB
VR = VOCAB + 8

NC = 2
NS = 16
NW = NC * NS
RPT = BATCH // NW
CH = 64
NCH = RPT // CH
CHW = CH * SEQ
NFULL = SEQ // 16
TAIL = SEQ - NFULL * 16


def _proj_body(w_ref, pb_ref, emb_ref, o_ref):
    i = pl.program_id(0)
    acc = lax.dot_general(
        w_ref[...], emb_ref[...],
        (((1,), (1,)), ((), ())),
        preferred_element_type=jnp.float32,
    )
    u0 = lax.bitcast_convert_type(
        acc[0:1, :].astype(jnp.bfloat16), jnp.uint16).astype(jnp.int32)
    u1 = lax.bitcast_convert_type(
        acc[1:2, :].astype(jnp.bfloat16), jnp.uint16).astype(jnp.int32)
    packed = u0 | (u1 << 16)
    v = i * VB + lax.broadcasted_iota(jnp.int32, (1, VB), 1)
    o_ref[...] = jnp.where(v < VOCAB, packed,
                           jnp.where(v == BIAS_IDX, pb_ref[...], 0))


def _project_table(emb_table, fc_w, fc_b):
    w8 = jnp.zeros((8, EMBED_DIM), jnp.float32).at[:2].set(fc_w * (1.0 / SEQ))
    bu = lax.bitcast_convert_type(
        fc_b.astype(jnp.bfloat16), jnp.uint16).astype(jnp.int32)
    pbias = bu[0] | (bu[1] << 16)
    pb_full = jnp.broadcast_to(pbias, (1, VB))
    return pl.pallas_call(
        _proj_body,
        grid=(VP // VB,),
        in_specs=[
            pl.BlockSpec((8, EMBED_DIM), lambda i: (0, 0)),
            pl.BlockSpec((1, VB), lambda i: (0, 0)),
            pl.BlockSpec((VB, EMBED_DIM), lambda i: (i, 0)),
        ],
        out_specs=pl.BlockSpec((1, VB), lambda i: (0, i)),
        out_shape=jax.ShapeDtypeStruct((1, VP), jnp.int32),
    )(w8, pb_full, emb_table).reshape(VP)


def _sc_pool_body(p_hbm, x_hbm, out_hbm,
                  p_v, xa_v, xb_v, o_v, sem_p, sem_a, sem_b):
    wid = lax.axis_index("s") * NC + lax.axis_index("c")
    row0 = wid * RPT

    cp_p = pltpu.async_copy(p_hbm.at[pl.ds(0, VR)], p_v, sem_p)
    xa_v[pl.ds(CHW, 16)] = jnp.zeros((16,), jnp.int32)
    xb_v[pl.ds(CHW, 16)] = jnp.zeros((16,), jnp.int32)

    bufs = (xa_v, xb_v)
    sems = (sem_a, sem_b)
    handles = [None] * NCH
    handles[0] = pltpu.async_copy(
        x_hbm.at[pl.ds(row0 * SEQ, CHW)], xa_v.at[pl.ds(0, CHW)], sem_a)

    cp_p.wait()
    bvec = plsc.load_gather(p_v, [jnp.full((16,), BIAS_IDX, jnp.int32)])
    b0s, b1s = plsc.unpack(plsc.bitcast(bvec, jnp.bfloat16),
                           format=plsc.PackFormat.INTERLEAVED)
    lane = lax.broadcasted_iota(jnp.int32, (16,), 0)
    tail_mask = lane < TAIL
    zf = jnp.zeros((16,), jnp.float32)

    for ch in range(NCH):
        if ch + 1 < NCH:
            handles[ch + 1] = pltpu.async_copy(
                x_hbm.at[pl.ds((row0 + (ch + 1) * CH) * SEQ, CHW)],
                bufs[(ch + 1) % 2].at[pl.ds(0, CHW)], sems[(ch + 1) % 2])
        handles[ch].wait()
        cur = bufs[ch % 2]

        def group_body(g, carry, cur=cur, ch=ch):
            def row_body(rr, vecs):
                vec0, vec1 = vecs
                off = (g * 16 + rr) * SEQ
                acc0 = zf
                acc1 = zf
                for j in range(NFULL):
                    gi = plsc.load_gather(p_v, [cur[pl.ds(off + j * 16, 16)]])
                    a, b = plsc.unpack(plsc.bitcast(gi, jnp.bfloat16),
                                       format=plsc.PackFormat.INTERLEAVED)
                    acc0 = acc0 + a
                    acc1 = acc1 + b
                gi = plsc.load_gather(p_v, [cur[pl.ds(off + NFULL * 16, 16)]])
                a, b = plsc.unpack(plsc.bitcast(gi, jnp.bfloat16),
                                   format=plsc.PackFormat.INTERLEAVED)
                acc0 = acc0 + jnp.where(tail_mask, a, 0.0)
                acc1 = acc1 + jnp.where(tail_mask, b, 0.0)
                return (jnp.where(lane == rr, jnp.sum(acc0), vec0),
                        jnp.where(lane == rr, jnp.sum(acc1), vec1))

            vec0, vec1 = lax.fori_loop(0, 16, row_body, (zf, zf))
            base = ch * CH + g * 16
            o_v[pl.ds(base, 16)] = vec0 + b0s
            o_v[pl.ds(RPT + base, 16)] = vec1 + b1s
            return carry

        lax.fori_loop(0, CH // 16, group_body, 0)

    pltpu.sync_copy(o_v.at[pl.ds(0, RPT)], out_hbm.at[0, pl.ds(row0, RPT)])
    pltpu.sync_copy(o_v.at[pl.ds(RPT, RPT)], out_hbm.at[1, pl.ds(row0, RPT)])


_sc_pool = functools.partial(
    pl.kernel,
    out_type=jax.ShapeDtypeStruct((2, BATCH), jnp.float32),
    mesh=plsc.VectorSubcoreMesh(
        core_axis_name="c", subcore_axis_name="s",
        num_cores=NC, num_subcores=NS,
    ),
    scratch_types=[
        pltpu.VMEM((VR,), jnp.int32),
        pltpu.VMEM((CHW + 16,), jnp.int32),
        pltpu.VMEM((CHW + 16,), jnp.int32),
        pltpu.VMEM((2 * RPT,), jnp.float32),
        pltpu.SemaphoreType.DMA,
        pltpu.SemaphoreType.DMA,
        pltpu.SemaphoreType.DMA,
    ],
    compiler_params=pltpu.CompilerParams(needs_layout_passes=False),
)(_sc_pool_body)


@jax.jit
def kernel(x, emb_table, fc_w, fc_b):
    p_packed = _project_table(emb_table, fc_w, fc_b)
    x_flat = x.astype(jnp.int32).reshape(BATCH * SEQ)
    out2 = _sc_pool(p_packed, x_flat)
    return out2.T
